# Initial kernel scaffold; baseline (speedup 1.0000x reference)
import jax
import jax.numpy as jnp
from jax.experimental import pallas as pl


def kernel(x, edge_index, batch, params):
    raise NotImplementedError("write your pallas kernel here")


# speedup vs baseline: 8.4635x; 8.4635x over previous
import functools

import jax
import jax.numpy as jnp
from jax import lax
from jax.experimental import pallas as pl
from jax.experimental.pallas import tpu as pltpu
from jax.experimental.pallas import tpu_sc as plsc

N = 10000
NGRAPH = 128
EMB = 128
HID = 256
REP = 128
NFEAT = 9
VOCAB = 64

NP = 10240
NCHUNK_N = NP // 128
NROWS = NP // 128
NROW_T = NP // 16

NW = 32
ECH = 128
CPT = 81
EPT = CPT * ECH
E_PAD = NW * EPT
DUMMY = N

F32 = jnp.float32
HIGH = lax.Precision.HIGHEST


def _zero2d(ref, nrows):
    def zbody(i, _):
        r = i // 8
        jcol = (i % 8) * 16
        ref[r, pl.ds(jcol, 16)] = jnp.zeros((16,), F32)
        return 0
    lax.fori_loop(0, nrows * 8, zbody, 0)


def _zero1d(ref, n):
    def zbody(i, _):
        ref[pl.ds(i * 16, 16)] = jnp.zeros((16,), F32)
        return 0
    lax.fori_loop(0, n // 16, zbody, 0)


def _enc_deg_body(xft_hbm, emb_hbm, dst_hbm, h0_hbm, degp_hbm,
                  idxb, rowb, dstb, degb, sem):
    c = lax.axis_index("c")
    s = lax.axis_index("s")
    wid = c * 16 + s

    _zero1d(degb, NP)

    for k in range(3):
        ci = wid + NW * k

        @pl.when(ci < NCHUNK_N)
        def _():
            off = ci * 128
            for f in range(NFEAT):
                pltpu.sync_copy(xft_hbm.at[pl.ds(f * NP + off, 128)], idxb)
                pltpu.async_copy(emb_hbm.at[idxb], rowb, sem,
                                 add=(f > 0)).wait()
            pltpu.sync_copy(rowb, h0_hbm.at[pl.ds(off, 128)])

    ones16 = jnp.ones((16,), F32)

    def dbody(i, _):
        goff = (wid * CPT + i) * ECH
        pltpu.sync_copy(dst_hbm.at[pl.ds(goff, ECH)], dstb)
        for j in range(8):
            d16 = dstb[pl.ds(j * 16, 16)]
            plsc.addupdate_scatter(degb, [d16], ones16)
        return 0
    lax.fori_loop(0, CPT, dbody, 0)

    pltpu.sync_copy(degb, degp_hbm.at[pl.ds(wid * NP, NP)])


def _enc_deg_call(xft, embf, dstp):
    fn = pl.kernel(
        _enc_deg_body,
        out_type=(jax.ShapeDtypeStruct((NP, 128), F32),
                  jax.ShapeDtypeStruct((NW * NP,), F32)),
        mesh=plsc.VectorSubcoreMesh(core_axis_name="c", subcore_axis_name="s"),
        compiler_params=pltpu.CompilerParams(needs_layout_passes=False),
        scratch_types=[
            pltpu.VMEM((128,), jnp.int32),
            pltpu.VMEM((128, 128), F32),
            pltpu.VMEM((ECH,), jnp.int32),
            pltpu.VMEM((NP,), F32),
            pltpu.SemaphoreType.DMA,
        ],
    )
    return fn(xft, embf, dstp)


def _softexp(x):
    y = jnp.maximum(x * 1.4426950408889634, -120.0)
    magic = 12582912.0
    n = (y + magic) - magic
    f = y - n
    t = f * 0.6931471805599453
    p = 1.0 + t * (1.0 + t * (0.5 + t * (
        0.16666666666666666 + t * (0.041666666666666664 + t * (
            0.008333333333333333 + t * 0.001388888888888889)))))
    ni = n.astype(jnp.int32)
    sc = plsc.bitcast(lax.shift_left(ni + 127, 23), jnp.float32)
    return p * sc


def _edge_body(halves, src_hbm, dst_hbm, asrc_hbm, adst_hbm, cvec_hbm, *rest):
    xws = rest[:halves]
    xgs = rest[halves:2 * halves]
    sp_hbm, pa_hbm, pb_hbm, exh_hbm = rest[2 * halves:2 * halves + 4]
    (abuf, bbuf, cbuf, exw, srcb, dstb, rowb, zbuf, sbuf,
     acc, sem) = rest[2 * halves + 4:]

    c = lax.axis_index("c")
    s = lax.axis_index("s")
    wid = c * 16 + s

    pltpu.sync_copy(asrc_hbm, abuf)
    pltpu.sync_copy(adst_hbm, bbuf)
    pltpu.sync_copy(cvec_hbm.at[pl.ds(0, 16)], cbuf)
    cshift = cbuf[pl.ds(0, 16)][0]

    _zero1d(sbuf, NP)
    _zero2d(zbuf, 8)

    def abody(i, _):
        goff = (wid * CPT + i) * ECH
        pltpu.sync_copy(src_hbm.at[pl.ds(goff, ECH)], srcb)
        pltpu.sync_copy(dst_hbm.at[pl.ds(goff, ECH)], dstb)
        for j in range(8):
            s16 = srcb[pl.ds(j * 16, 16)]
            d16 = dstb[pl.ds(j * 16, 16)]
            av = plsc.load_gather(abuf, [s16])
            dv = plsc.load_gather(bbuf, [d16])
            e = av + dv
            e = jnp.maximum(e, 0.0) + 0.2 * jnp.minimum(e, 0.0)
            ex = _softexp(e - cshift)
            exw[pl.ds(j * 16, 16)] = ex
            plsc.addupdate_scatter(sbuf, [d16], ex)
        pltpu.sync_copy(exw, exh_hbm.at[pl.ds(goff, ECH)])
        return 0
    lax.fori_loop(0, CPT, abody, 0)

    pltpu.sync_copy(sbuf, sp_hbm.at[pl.ds(wid * NP, NP)])

    def run_half(feat_hbm, out_hbm, h_idx, weighted):
        def zacc(z, _):
            pltpu.sync_copy(zbuf, acc.at[pl.ds(s * NROW_T + z * 8, 8)])
            return 0
        lax.fori_loop(0, NROW_T // 8, zacc, 0)
        plsc.subcore_barrier()

        def bbody(i, _):
            goff = (wid * CPT + i) * ECH
            pltpu.sync_copy(src_hbm.at[pl.ds(goff, ECH)], srcb)
            pltpu.sync_copy(dst_hbm.at[pl.ds(goff, ECH)], dstb)
            pltpu.async_copy(feat_hbm.at[srcb], rowb, sem).wait()
            if weighted:
                pltpu.sync_copy(exh_hbm.at[pl.ds(goff, ECH)], exw)

                def wbody(g, _):
                    exv = exw[pl.ds(g * 16, 16)]
                    for e in range(16):
                        wv = lax.broadcast(exv[e], (16,))
                        r = g * 16 + e
                        for j in range(8):
                            rowb[r, pl.ds(j * 16, 16)] = (
                                rowb[r, pl.ds(j * 16, 16)] * wv)
                    return 0
                lax.fori_loop(0, ECH // 16, wbody, 0)
            pltpu.sync_copy(rowb, acc.at[dstb], add=True)
            return 0
        lax.fori_loop(0, CPT, bbody, 0)
        plsc.subcore_barrier()
        pltpu.sync_copy(acc.at[pl.ds(s * NROW_T, NROW_T)],
                        out_hbm.at[c, h_idx, pl.ds(s * NROW_T, NROW_T)])
        plsc.subcore_barrier()

    for h in range(halves):
        run_half(xws[h], pa_hbm, h, False)
    for h in range(halves):
        run_half(xgs[h], pb_hbm, h, True)


def _edge_call(halves, srcp, dstp, asrc, adst, cvec, xws, xgs):
    fn = pl.kernel(
        functools.partial(_edge_body, halves),
        out_type=(jax.ShapeDtypeStruct((NW * NP,), F32),
                  jax.ShapeDtypeStruct((2, halves, NP, 128), F32),
                  jax.ShapeDtypeStruct((2, halves, NP, 128), F32),
                  jax.ShapeDtypeStruct((E_PAD,), F32)),
        mesh=plsc.VectorSubcoreMesh(core_axis_name="c", subcore_axis_name="s"),
        compiler_params=pltpu.CompilerParams(needs_layout_passes=False),
        scratch_types=[
            pltpu.VMEM((NP,), F32),
            pltpu.VMEM((NP,), F32),
            pltpu.VMEM((16,), F32),
            pltpu.VMEM((ECH,), F32),
            pltpu.VMEM((ECH,), jnp.int32),
            pltpu.VMEM((ECH,), jnp.int32),
            pltpu.VMEM((ECH, 128), F32),
            pltpu.VMEM((8, 128), F32),
            pltpu.VMEM((NP,), F32),
            pltpu.VMEM_SHARED((NP, 128), F32),
            pltpu.SemaphoreType.DMA,
        ],
    )
    return fn(srcp, dstp, asrc, adst, cvec, *xws, *xgs)[:3]


def _mm_outputs(h_blk, dinv_blk, wg_ref, wa_ref, ats_ref, atd_ref,
                outs, mx_ref, i, dout):
    halves = dout // 128
    xw = jnp.dot(h_blk, wg_ref[...])
    xws = xw * dinv_blk
    xg = jnp.dot(h_blk, wa_ref[...])
    for h in range(halves):
        outs["xw"][h][...] = xws[:, h * 128:(h + 1) * 128]
        outs["xg"][h][...] = xg[:, h * 128:(h + 1) * 128]
    a_s = jnp.dot(xg, ats_ref[...])
    a_d = jnp.dot(xg, atd_ref[...])
    outs["asrc"][...] = a_s
    outs["adst"][...] = a_d

    @pl.when(i == 0)
    def _():
        mx_ref[0] = -jnp.inf
        mx_ref[1] = -jnp.inf
    mx_ref[0] = jnp.maximum(mx_ref[0], jnp.max(a_s))
    mx_ref[1] = jnp.maximum(mx_ref[1], jnp.max(a_d))

    @pl.when(i == NCHUNK_N - 1)
    def _():
        cfin = jnp.maximum(mx_ref[0] + mx_ref[1], 0.0)
        outs["c"][...] = jnp.full((1, 128), cfin, F32)


def _tc0_body(h0_ref, degp_ref, wg_ref, wa_ref, ats_ref, atd_ref,
              xw0, xw1, xg0, xg1, asrc, adst, cout, dinv_out, mx_ref):
    i = pl.program_id(0)
    deg = jnp.sum(degp_ref[...], axis=0)
    dinv = lax.rsqrt(jnp.maximum(deg, 1e-12))
    dinv_out[...] = dinv[:, None]
    outs = {"xw": [xw0, xw1], "xg": [xg0, xg1],
            "asrc": asrc, "adst": adst, "c": cout}
    _mm_outputs(h0_ref[...], dinv[:, None], wg_ref, wa_ref, ats_ref, atd_ref,
                outs, mx_ref, i, HID)


def _tc0_call(h0, degp, wg, wa, ats, atd):
    out_shape = [
        jax.ShapeDtypeStruct((NP, 128), F32),
        jax.ShapeDtypeStruct((NP, 128), F32),
        jax.ShapeDtypeStruct((NP, 128), F32),
        jax.ShapeDtypeStruct((NP, 128), F32),
        jax.ShapeDtypeStruct((NP, 1), F32),
        jax.ShapeDtypeStruct((NP, 1), F32),
        jax.ShapeDtypeStruct((1, 128), F32),
        jax.ShapeDtypeStruct((NP, 1), F32),
    ]
    grid = (NCHUNK_N,)
    full = lambda i: (0, 0)
    row_spec = pl.BlockSpec((128, 128), lambda i: (i, 0))
    col1 = pl.BlockSpec((128, 1), lambda i: (i, 0))
    return pl.pallas_call(
        _tc0_body,
        grid=grid,
        in_specs=[
            pl.BlockSpec((128, EMB), lambda i: (i, 0)),
            pl.BlockSpec((NW, 128), lambda i: (0, i)),
            pl.BlockSpec((EMB, HID), full),
            pl.BlockSpec((EMB, HID), full),
            pl.BlockSpec((HID, 1), full),
            pl.BlockSpec((HID, 1), full),
        ],
        out_specs=[row_spec, row_spec, row_spec, row_spec, col1, col1,
                   pl.BlockSpec((1, 128), full), col1],
        out_shape=out_shape,
        scratch_shapes=[pltpu.SMEM((2,), F32)],
    )(h0, degp, wg, wa, ats, atd)


def _combine(pa_ref, pb_ref, sp_ref, dinv_blk, bg_ref, ba_ref, th_ref, dprev):
    hprev = dprev // 128
    x_a = jnp.concatenate(
        [pa_ref[0, h] + pa_ref[1, h] for h in range(hprev)], axis=1)
    x_b = jnp.concatenate(
        [pb_ref[0, h] + pb_ref[1, h] for h in range(hprev)], axis=1)
    sseg = jnp.sum(sp_ref[...], axis=0)
    x_a = x_a * dinv_blk + bg_ref[...]
    x_b = x_b / (sseg + 1e-16)[:, None] + ba_ref[...]
    th = jnp.clip(th_ref[...], 0.0, 1.0)
    return jnp.maximum(x_a * th + x_b * (1.0 - th), 0.0)


def _tcmid_body(dprev, dout, pa_ref, pb_ref, sp_ref, dinv_ref,
                bg_ref, ba_ref, th_ref, wg_ref, wa_ref, ats_ref, atd_ref,
                *rest):
    halves = dout // 128
    outs_flat = rest[:2 * halves + 3]
    mx_ref = rest[2 * halves + 3]
    i = pl.program_id(0)
    h_blk = _combine(pa_ref, pb_ref, sp_ref, dinv_ref[...], bg_ref, ba_ref,
                     th_ref, dprev)
    outs = {"xw": list(outs_flat[:halves]),
            "xg": list(outs_flat[halves:2 * halves]),
            "asrc": outs_flat[2 * halves],
            "adst": outs_flat[2 * halves + 1],
            "c": outs_flat[2 * halves + 2]}
    _mm_outputs(h_blk, dinv_ref[...], wg_ref, wa_ref, ats_ref, atd_ref,
                outs, mx_ref, i, dout)


def _tcmid_call(dprev, dout, pa, pb, sp, dinv, bg, ba, th, wg, wa, ats, atd):
    halves = dout // 128
    hprev = dprev // 128
    grid = (NCHUNK_N,)
    full = lambda i: (0, 0)
    row_spec = pl.BlockSpec((128, 128), lambda i: (i, 0))
    col1 = pl.BlockSpec((128, 1), lambda i: (i, 0))
    out_shape = ([jax.ShapeDtypeStruct((NP, 128), F32)] * (2 * halves) +
                 [jax.ShapeDtypeStruct((NP, 1), F32),
                  jax.ShapeDtypeStruct((NP, 1), F32),
                  jax.ShapeDtypeStruct((1, 128), F32)])
    return pl.pallas_call(
        functools.partial(_tcmid_body, dprev, dout),
        grid=grid,
        in_specs=[
            pl.BlockSpec((2, hprev, 128, 128), lambda i: (0, 0, i, 0)),
            pl.BlockSpec((2, hprev, 128, 128), lambda i: (0, 0, i, 0)),
            pl.BlockSpec((NW, 128), lambda i: (0, i)),
            col1,
            pl.BlockSpec((1, dprev), full),
            pl.BlockSpec((1, dprev), full),
            pl.BlockSpec((1, dprev), full),
            pl.BlockSpec((dprev, dout), full),
            pl.BlockSpec((dprev, dout), full),
            pl.BlockSpec((dout, 1), full),
            pl.BlockSpec((dout, 1), full),
        ],
        out_specs=([row_spec] * (2 * halves) +
                   [col1, col1, pl.BlockSpec((1, 128), full)]),
        out_shape=out_shape,
        scratch_shapes=[pltpu.SMEM((2,), F32)],
    )(pa, pb, sp, dinv, bg, ba, th, wg, wa, ats, atd)


def _pool_body(pa_ref, pb_ref, sp_ref, dinv_ref, bg_ref, ba_ref, th_ref,
               batch_ref, lw_ref, lb_ref, out_ref, p_ref, cnt_ref):
    i = pl.program_id(0)
    h_blk = _combine(pa_ref, pb_ref, sp_ref, dinv_ref[...], bg_ref, ba_ref,
                     th_ref, REP)

    @pl.when(i == 0)
    def _():
        p_ref[...] = jnp.zeros((NGRAPH, REP), F32)
        cnt_ref[...] = jnp.zeros((NGRAPH, 1), F32)

    gids = lax.broadcasted_iota(jnp.int32, (128, NGRAPH), 1)
    onehot = (gids == batch_ref[...]).astype(F32)
    dn = (((0,), (0,)), ((), ()))
    p_ref[...] += lax.dot_general(onehot, h_blk, dn, precision=HIGH)
    cnt_ref[...] += lax.dot_general(onehot, jnp.ones((128, 1), F32), dn,
                                    precision=HIGH)

    @pl.when(i == NCHUNK_N - 1)
    def _():
        pooled = p_ref[...] / jnp.maximum(cnt_ref[...], 1.0)
        out_ref[...] = jnp.dot(pooled, lw_ref[...]) + lb_ref[...]


def _pool_call(pa, pb, sp, dinv, bg, ba, th, batch_p, lw, lb):
    grid = (NCHUNK_N,)
    full = lambda i: (0, 0)
    col1 = pl.BlockSpec((128, 1), lambda i: (i, 0))
    return pl.pallas_call(
        _pool_body,
        grid=grid,
        in_specs=[
            pl.BlockSpec((2, 1, 128, 128), lambda i: (0, 0, i, 0)),
            pl.BlockSpec((2, 1, 128, 128), lambda i: (0, 0, i, 0)),
            pl.BlockSpec((NW, 128), lambda i: (0, i)),
            col1,
            pl.BlockSpec((1, REP), full),
            pl.BlockSpec((1, REP), full),
            pl.BlockSpec((1, REP), full),
            pl.BlockSpec((128, 1), lambda i: (i, 0)),
            pl.BlockSpec((REP, 1), full),
            pl.BlockSpec((1, 1), full),
        ],
        out_specs=pl.BlockSpec((NGRAPH, 1), full),
        out_shape=jax.ShapeDtypeStruct((NGRAPH, 1), F32),
        scratch_shapes=[pltpu.VMEM((NGRAPH, REP), F32),
                        pltpu.VMEM((NGRAPH, 1), F32)],
    )(pa, pb, sp, dinv, bg, ba, th, batch_p, lw, lb)


def kernel(x, edge_index, batch, params):
    feat_off = jnp.arange(NFEAT, dtype=jnp.int32)[:, None] * VOCAB
    xft = feat_off + x.T.astype(jnp.int32)
    xft = jnp.pad(xft, ((0, 0), (0, NP - N)),
                  constant_values=NFEAT * VOCAB)
    xft = xft.reshape(NFEAT * NP)
    embf = params["emb_tables"].reshape(NFEAT * VOCAB, EMB)
    embf = jnp.concatenate([embf, jnp.zeros((8, EMB), F32)], axis=0)

    loop = jnp.arange(N, dtype=jnp.int32)
    srcp = jnp.concatenate([edge_index[0].astype(jnp.int32), loop])
    dstp = jnp.concatenate([edge_index[1].astype(jnp.int32), loop])
    srcp = jnp.pad(srcp, (0, E_PAD - srcp.shape[0]), constant_values=DUMMY)
    dstp = jnp.pad(dstp, (0, E_PAD - dstp.shape[0]), constant_values=DUMMY)

    batch_p = jnp.pad(batch.astype(jnp.int32), (0, NP - N),
                      constant_values=NGRAPH)[:, None]

    def att(i):
        return (params["att_src%d" % i][:, None],
                params["att_dst%d" % i][:, None])

    def lay(i):
        return (params["gcn_b%d" % i][None, :],
                params["gat_b%d" % i][None, :],
                params["theta%d" % i][None, :])

    h0, degp = _enc_deg_call(xft, embf, dstp)

    ats, atd = att(0)
    xw0, xw1, xg0, xg1, asrc, adst, cvec, dinv = _tc0_call(
        h0, degp.reshape(NW, NP), params["gcn_W0"], params["gat_W0"],
        ats, atd)
    sp, pa, pb = _edge_call(2, srcp, dstp, asrc.reshape(NP),
                            adst.reshape(NP), cvec.reshape(128),
                            [xw0, xw1], [xg0, xg1])

    for i in (1, 2, 3):
        dout = 128 if i == 3 else 256
        halves = dout // 128
        bg, ba, th = lay(i - 1)
        ats, atd = att(i)
        res = _tcmid_call(256, dout, pa, pb, sp.reshape(NW, NP), dinv,
                          bg, ba, th,
                          params["gcn_W%d" % i], params["gat_W%d" % i],
                          ats, atd)
        xws = list(res[:halves])
        xgs = list(res[halves:2 * halves])
        asrc, adst, cvec = res[2 * halves:]
        sp, pa, pb = _edge_call(halves, srcp, dstp, asrc.reshape(NP),
                                adst.reshape(NP), cvec.reshape(128),
                                xws, xgs)

    bg, ba, th = lay(3)
    return _pool_call(pa, pb, sp.reshape(NW, NP), dinv, bg, ba, th, batch_p,
                      params["lin_W"], params["lin_b"][None, :])
